# baseline (speedup 1.0000x reference)
import functools

import jax
import jax.numpy as jnp
from jax import lax
from jax.experimental import pallas as pl
from jax.experimental.pallas import tpu as pltpu
from jax.experimental.pallas import tpu_sc as plsc

B = 16384
DIM = 64
RANK = 8
HRANGE = 2 ** 14
OUT_BITS = 14
C0 = 2654435761
C1 = 2246822519
C2 = 3266489917

NC = 2
NS = 16
NW = NC * NS
EPW = B // NW
GRP = 16
NGRP = EPW // GRP
NPAIR = NGRP // 2
DB = 8
NDB = DIM // DB
ROW = DIM * RANK
RR = RANK * RANK
MR = GRP


def _tt_body(x_hbm, s_hbm, e_hbm, m_hbm, out_hbm,
             xbuf, img_a, img_b, imc, ie_a, is_a, ie_b, is_b,
             mb0, mb1, eb_a, sb_a, eb_b, sb_b, outb,
             sem_m0, sem_m1, sem_ea, sem_eb):
    cid = lax.axis_index("c")
    sid = lax.axis_index("s")
    wid = cid * NS + sid
    base = wid * EPW
    pltpu.sync_copy(x_hbm.at[pl.ds(base, EPW)], xbuf)
    lanes = lax.iota(jnp.int32, GRP)
    mb = (mb0, mb1)
    sem_m = (sem_m0, sem_m1)

    def hashes(g):
        xv = xbuf[pl.ds(g * GRP, GRP)]
        xu = xv.astype(jnp.uint32)
        sh = jnp.uint32(32 - OUT_BITS)
        h0 = lax.shift_right_logical(xu * jnp.uint32(C0), sh).astype(jnp.int32)
        h1 = lax.shift_right_logical(xu * jnp.uint32(C1), sh).astype(jnp.int32)
        h2 = lax.shift_right_logical(xu * jnp.uint32(C2), sh).astype(jnp.int32)
        return h0, h1, h2

    def stage_group(g, img, ie, isx, eb, sb, sem_es):
        h0, h1, h2 = hashes(g)
        ie[...] = h2
        isx[...] = h0
        img[...] = h1 * NDB
        pltpu.async_copy(e_hbm.at[ie], eb, sem_es)
        pltpu.async_copy(s_hbm.at[isx], sb, sem_es)

    def fire_m(img, db, par):
        rows = img[...] + db
        imc[pl.ds(par * GRP, GRP)] = rows
        pltpu.async_copy(m_hbm.at[imc.at[pl.ds(par * GRP, GRP)]],
                         mb[par], sem_m[par])

    def drain_m(par):
        pltpu.make_async_copy(m_hbm.at[pl.ds(0, MR)], mb[par],
                              sem_m[par]).wait()


    def drain_es(eb, sb, sem_es):
        pltpu.make_async_copy(e_hbm.at[pl.ds(0, GRP)], eb, sem_es).wait()
        pltpu.make_async_copy(s_hbm.at[pl.ds(0, GRP)], sb, sem_es).wait()

    def compute_group(g, eb, sb):
        def dblock_db(db):
            def dloop(dd, c2, db=db):
                d = db * DB + dd
                colbase = d * RANK
                mcv = jnp.full((GRP,), dd * RR, jnp.int32)
                sv = [plsc.load_gather(
                          sb, [lanes, jnp.full((GRP,), colbase + s, jnp.int32)])
                      for s in range(RANK)]
                vv = [plsc.load_gather(
                          eb, [lanes, jnp.full((GRP,), colbase + r, jnp.int32)])
                      for r in range(RANK)]
                acc = jnp.zeros((GRP,), jnp.float32)
                for r in range(RANK):
                    t = jnp.zeros((GRP,), jnp.float32)
                    for s in range(RANK):
                        m = plsc.load_gather(
                            mb[db % 2], [lanes, mcv + (r * RANK + s)])
                        t = t + m * sv[s]
                    acc = acc + t * vv[r]
                plsc.store_scatter(
                    outb, [lanes, jnp.full((GRP,), d, jnp.int32)], acc)
                return c2

            lax.fori_loop(0, DB, dloop, 0)

        return dblock_db

    stage_group(0, img_a, ie_a, is_a, eb_a, sb_a, sem_ea)
    fire_m(img_a, 0, 0)

    def pair_body(gp, carry):
        g = 2 * gp
        drain_es(eb_a, sb_a, sem_ea)
        stage_group(g + 1, img_b, ie_b, is_b, eb_b, sb_b, sem_eb)
        dblk = compute_group(g, eb_a, sb_a)
        for db in range(NDB):
            drain_m(db % 2)
            if db < NDB - 1:
                fire_m(img_a, db + 1, (db + 1) % 2)
            else:
                fire_m(img_b, 0, (db + 1) % 2)
            dblk(db)
        pltpu.sync_copy(outb, out_hbm.at[pl.ds(base + g * GRP, GRP), :])

        drain_es(eb_b, sb_b, sem_eb)

        @pl.when(gp + 1 < NPAIR)
        def _():
            stage_group(g + 2, img_a, ie_a, is_a, eb_a, sb_a, sem_ea)

        dblk = compute_group(g + 1, eb_b, sb_b)
        for db in range(NDB):
            drain_m(db % 2)
            if db < NDB - 1:
                fire_m(img_b, db + 1, (db + 1) % 2)
            else:
                @pl.when(gp + 1 < NPAIR)
                def _():
                    fire_m(img_a, 0, (db + 1) % 2)
            dblk(db)
        pltpu.sync_copy(outb, out_hbm.at[pl.ds(base + (g + 1) * GRP, GRP), :])
        return carry

    lax.fori_loop(0, NPAIR, pair_body, 0)


@jax.jit
def _tt_embed(x, s2, e2, m2):
    mesh = plsc.VectorSubcoreMesh(core_axis_name="c", subcore_axis_name="s")
    f = functools.partial(
        pl.kernel,
        out_type=jax.ShapeDtypeStruct((B, DIM), jnp.float32),
        mesh=mesh,
        scratch_types=[
            pltpu.VMEM((EPW,), jnp.int32),
            pltpu.VMEM((GRP,), jnp.int32),
            pltpu.VMEM((GRP,), jnp.int32),
            pltpu.VMEM((2 * GRP,), jnp.int32),
            pltpu.VMEM((GRP,), jnp.int32),
            pltpu.VMEM((GRP,), jnp.int32),
            pltpu.VMEM((GRP,), jnp.int32),
            pltpu.VMEM((GRP,), jnp.int32),
            pltpu.VMEM((MR, ROW), jnp.float32),
            pltpu.VMEM((MR, ROW), jnp.float32),
            pltpu.VMEM((GRP, ROW), jnp.float32),
            pltpu.VMEM((GRP, ROW), jnp.float32),
            pltpu.VMEM((GRP, ROW), jnp.float32),
            pltpu.VMEM((GRP, ROW), jnp.float32),
            pltpu.VMEM((GRP, DIM), jnp.float32),
            pltpu.SemaphoreType.DMA,
            pltpu.SemaphoreType.DMA,
            pltpu.SemaphoreType.DMA,
            pltpu.SemaphoreType.DMA,
        ],
        compiler_params=pltpu.CompilerParams(
            use_tc_tiling_on_sc=False, needs_layout_passes=False,
            disable_semaphore_checks=True),
    )(_tt_body)
    return f(x, s2, e2, m2)


def kernel(x, start_core, end_core, cores):
    s2 = start_core.reshape(HRANGE, ROW)
    e2 = end_core.reshape(HRANGE, ROW)
    m2 = cores.reshape(HRANGE * NDB, ROW)
    return _tt_embed(x, s2, e2, m2)


# speedup vs baseline: 1.8483x; 1.8483x over previous
import functools

import jax
import jax.numpy as jnp
from jax import lax
from jax.experimental import pallas as pl
from jax.experimental.pallas import tpu as pltpu
from jax.experimental.pallas import tpu_sc as plsc

B = 16384
DIM = 64
RANK = 8
HRANGE = 2 ** 14
OUT_BITS = 14
C0 = 2654435761
C1 = 2246822519
C2 = 3266489917

NC = 2
NS = 16
NW = NC * NS
EPW = B // NW
GRP = 16
NGRP = EPW // GRP
NPAIR = NGRP // 2
DB = 8
NDB = DIM // DB
ROW = DIM * RANK
RR = RANK * RANK
MR = DB * GRP


def _tt_body(x_hbm, s_hbm, e_hbm, m_hbm, out_hbm,
             xbuf, img_a, img_b, ie_a, is_a, ie_b, is_b,
             mb0, mb1, eb_a, sb_a, eb_b, sb_b, outb,
             sem_m0, sem_m1, sem_ea, sem_eb):
    cid = lax.axis_index("c")
    sid = lax.axis_index("s")
    wid = cid * NS + sid
    base = wid * EPW
    pltpu.sync_copy(x_hbm.at[pl.ds(base, EPW)], xbuf)
    lanes = lax.iota(jnp.int32, GRP)
    mb = (mb0, mb1)
    sem_m = (sem_m0, sem_m1)

    def hashes(g):
        xv = xbuf[pl.ds(g * GRP, GRP)]
        xu = xv.astype(jnp.uint32)
        sh = jnp.uint32(32 - OUT_BITS)
        h0 = lax.shift_right_logical(xu * jnp.uint32(C0), sh).astype(jnp.int32)
        h1 = lax.shift_right_logical(xu * jnp.uint32(C1), sh).astype(jnp.int32)
        h2 = lax.shift_right_logical(xu * jnp.uint32(C2), sh).astype(jnp.int32)
        return h0, h1, h2

    def stage_group(g, img, ie, isx, eb, sb, sem_es):
        h0, h1, h2 = hashes(g)
        ie[...] = h2
        isx[...] = h0
        h1d = h1 * DIM

        def fill(d, c):
            img[pl.ds(d * GRP, GRP)] = h1d + d
            return c

        lax.fori_loop(0, DIM, fill, 0)
        pltpu.async_copy(e_hbm.at[ie], eb, sem_es)
        pltpu.async_copy(s_hbm.at[isx], sb, sem_es)

    def fire_m(img, db, par):
        pltpu.async_copy(m_hbm.at[img.at[pl.ds(db * MR, MR)]],
                         mb[par], sem_m[par])

    def drain_m(par):
        pltpu.make_async_copy(m_hbm.at[pl.ds(0, MR)], mb[par],
                              sem_m[par]).wait()

    def drain_es(eb, sb, sem_es):
        pltpu.make_async_copy(e_hbm.at[pl.ds(0, GRP)], eb, sem_es).wait()
        pltpu.make_async_copy(s_hbm.at[pl.ds(0, GRP)], sb, sem_es).wait()

    def compute_group(g, eb, sb):
        def dblock_db(db):
            def dloop(dd, c2, db=db):
                d = db * DB + dd
                colbase = d * RANK
                rowv = dd * GRP + lanes
                sv = [plsc.load_gather(
                          sb, [lanes, jnp.full((GRP,), colbase + s, jnp.int32)])
                      for s in range(RANK)]
                vv = [plsc.load_gather(
                          eb, [lanes, jnp.full((GRP,), colbase + r, jnp.int32)])
                      for r in range(RANK)]
                acc = jnp.zeros((GRP,), jnp.float32)
                for r in range(RANK):
                    t = jnp.zeros((GRP,), jnp.float32)
                    for s in range(RANK):
                        m = plsc.load_gather(
                            mb[db % 2],
                            [rowv, jnp.full((GRP,), r * RANK + s, jnp.int32)])
                        t = t + m * sv[s]
                    acc = acc + t * vv[r]
                plsc.store_scatter(
                    outb, [lanes, jnp.full((GRP,), d, jnp.int32)], acc)
                return c2

            lax.fori_loop(0, DB, dloop, 0)

        return dblock_db

    stage_group(0, img_a, ie_a, is_a, eb_a, sb_a, sem_ea)
    fire_m(img_a, 0, 0)

    def pair_body(gp, carry):
        g = 2 * gp
        drain_es(eb_a, sb_a, sem_ea)
        stage_group(g + 1, img_b, ie_b, is_b, eb_b, sb_b, sem_eb)
        dblk = compute_group(g, eb_a, sb_a)
        for db in range(NDB):
            drain_m(db % 2)
            if db < NDB - 1:
                fire_m(img_a, db + 1, (db + 1) % 2)
            else:
                fire_m(img_b, 0, (db + 1) % 2)
            dblk(db)
        pltpu.sync_copy(outb, out_hbm.at[pl.ds(base + g * GRP, GRP), :])

        drain_es(eb_b, sb_b, sem_eb)

        @pl.when(gp + 1 < NPAIR)
        def _():
            stage_group(g + 2, img_a, ie_a, is_a, eb_a, sb_a, sem_ea)

        dblk = compute_group(g + 1, eb_b, sb_b)
        for db in range(NDB):
            drain_m(db % 2)
            if db < NDB - 1:
                fire_m(img_b, db + 1, (db + 1) % 2)
            else:
                @pl.when(gp + 1 < NPAIR)
                def _():
                    fire_m(img_a, 0, (db + 1) % 2)
            dblk(db)
        pltpu.sync_copy(outb, out_hbm.at[pl.ds(base + (g + 1) * GRP, GRP), :])
        return carry

    lax.fori_loop(0, NPAIR, pair_body, 0)


@jax.jit
def _tt_embed(x, s2, e2, m2):
    mesh = plsc.VectorSubcoreMesh(core_axis_name="c", subcore_axis_name="s")
    f = functools.partial(
        pl.kernel,
        out_type=jax.ShapeDtypeStruct((B, DIM), jnp.float32),
        mesh=mesh,
        scratch_types=[
            pltpu.VMEM((EPW,), jnp.int32),
            pltpu.VMEM((DIM * GRP,), jnp.int32),
            pltpu.VMEM((DIM * GRP,), jnp.int32),
            pltpu.VMEM((GRP,), jnp.int32),
            pltpu.VMEM((GRP,), jnp.int32),
            pltpu.VMEM((GRP,), jnp.int32),
            pltpu.VMEM((GRP,), jnp.int32),
            pltpu.VMEM((MR, RR), jnp.float32),
            pltpu.VMEM((MR, RR), jnp.float32),
            pltpu.VMEM((GRP, ROW), jnp.float32),
            pltpu.VMEM((GRP, ROW), jnp.float32),
            pltpu.VMEM((GRP, ROW), jnp.float32),
            pltpu.VMEM((GRP, ROW), jnp.float32),
            pltpu.VMEM((GRP, DIM), jnp.float32),
            pltpu.SemaphoreType.DMA,
            pltpu.SemaphoreType.DMA,
            pltpu.SemaphoreType.DMA,
            pltpu.SemaphoreType.DMA,
        ],
        compiler_params=pltpu.CompilerParams(
            use_tc_tiling_on_sc=False, needs_layout_passes=False,
            disable_semaphore_checks=True),
    )(_tt_body)
    return f(x, s2, e2, m2)


def kernel(x, start_core, end_core, cores):
    s2 = start_core.reshape(HRANGE, ROW)
    e2 = end_core.reshape(HRANGE, ROW)
    m2 = cores.reshape(HRANGE * DIM, RR)
    return _tt_embed(x, s2, e2, m2)
